# baseline (speedup 1.0000x reference)
import functools

import jax
import jax.numpy as jnp
import numpy as np
from jax import lax
from jax.experimental import pallas as pl
from jax.experimental.pallas import tpu as pltpu
from jax.experimental.pallas import tpu_sc as plsc

COEF_TEMP = 0.0001
N = 1048576
K = N // 2
NSUB = 16
CHUNK = N // NSUB
LANES = 16
NVREG = CHUNK // LANES
NPAIR = 15

_TOP = np.uint32(0x80000000)
_ALL = np.uint32(0xFFFFFFFF)


def _mesh():
    return plsc.VectorSubcoreMesh(
        core_axis_name="c", subcore_axis_name="s", num_cores=1)


@functools.partial(
    pl.kernel,
    out_type=jax.ShapeDtypeStruct((LANES,), jnp.float32),
    mesh=_mesh(),
    compiler_params=pltpu.CompilerParams(needs_layout_passes=False),
    scratch_types=[
        pltpu.VMEM((CHUNK,), jnp.float32),
        pltpu.VMEM((LANES,), jnp.float32),
        pltpu.VMEM((LANES,), jnp.int32),
        pltpu.VMEM((2 * NSUB, LANES), jnp.float32),
        pltpu.VMEM((NSUB, LANES), jnp.int32),
        pltpu.VMEM_SHARED((2 * NSUB, LANES), jnp.float32),
        pltpu.VMEM_SHARED((NSUB, LANES), jnp.int32),
    ],
)
def _sc_loss(inv_t_hbm, adv_hbm, out_hbm,
             data, stf, sti, rdf, rdi, shf, shi):
    sid = lax.axis_index("s")
    base = sid * CHUNK

    pltpu.sync_copy(adv_hbm.at[pl.ds(base, CHUNK)], data)
    pltpu.sync_copy(inv_t_hbm, stf)
    it_vec = stf[...]

    zi = jnp.zeros((LANES,), jnp.int32)
    zf = jnp.zeros((LANES,), jnp.float32)
    idx = lax.iota(jnp.int32, 16)

    def max_conv_body(i, acc):
        m0, m1, c0, c1 = acc
        for j in range(4):
            sl = pl.ds((4 * i + j) * LANES, LANES)
            x = data[sl]
            if j % 2 == 0:
                m0 = jnp.maximum(m0, x)
            else:
                m1 = jnp.maximum(m1, x)
            b = lax.bitcast_convert_type(x, jnp.int32)
            ku = lax.bitcast_convert_type(b, jnp.uint32) ^ jnp.where(
                b < 0, _ALL, _TOP)
            data[sl] = lax.bitcast_convert_type(ku, jnp.float32)
            if j % 2 == 0:
                c0 = c0 + jnp.where(ku >= _TOP, 1, 0).astype(jnp.int32)
            else:
                c1 = c1 + jnp.where(ku >= _TOP, 1, 0).astype(jnp.int32)
        return m0, m1, c0, c1

    ninf = jnp.full((LANES,), -jnp.inf, jnp.float32)
    m0, m1, c0, c1 = lax.fori_loop(
        0, NVREG // 4, max_conv_body, (ninf, ninf, zi, zi))
    stf[...] = jnp.maximum(m0, m1)
    sti[...] = jnp.full((LANES,), jnp.sum(c0 + c1), jnp.int32)
    pltpu.sync_copy(stf, shf.at[sid])
    pltpu.sync_copy(sti, shi.at[sid])
    plsc.subcore_barrier()
    pltpu.sync_copy(shf.at[pl.ds(0, NSUB)], rdf.at[pl.ds(0, NSUB)])
    pltpu.sync_copy(shi, rdi)

    def comb(i, carry):
        m, c = carry
        return jnp.maximum(m, rdf[i]), c + rdi[i]
    Mvec_all, cpos = lax.fori_loop(0, NSUB, comb, (ninf, zi))
    M = jnp.max(Mvec_all)
    Mvec = jnp.full((LANES,), M)
    tot_pos = jnp.max(cpos)
    lo0 = jnp.where(tot_pos >= K, jnp.uint32(_TOP), jnp.uint32(0))
    plsc.subcore_barrier()

    def round_body(r, lo):
        bit_a = jnp.uint32(30) - 2 * r.astype(jnp.uint32)
        bit_b = bit_a - 1
        one = jnp.uint32(1)
        cand_a = lo | (one << bit_a)
        cand_b0 = lo | (one << bit_b)
        cand_b1 = cand_a | (one << bit_b)
        va = jnp.full((LANES,), cand_a)
        vb0 = jnp.full((LANES,), cand_b0)
        vb1 = jnp.full((LANES,), cand_b1)

        def count_body(i, acc):
            aa, ab0, ab1 = acc
            for j in range(8):
                sl = pl.ds((8 * i + j) * LANES, LANES)
                ku = lax.bitcast_convert_type(data[sl], jnp.uint32)
                aa = aa + jnp.where(ku >= va, 1, 0).astype(jnp.int32)
                ab0 = ab0 + jnp.where(ku >= vb0, 1, 0).astype(jnp.int32)
                ab1 = ab1 + jnp.where(ku >= vb1, 1, 0).astype(jnp.int32)
            return aa, ab0, ab1

        aa, ab0, ab1 = lax.fori_loop(0, NVREG // 8, count_body, (zi, zi, zi))
        n_a = jnp.sum(aa)
        n_b0 = jnp.sum(ab0)
        n_b1 = jnp.sum(ab1)
        packed = jnp.where(
            idx < 6, n_a, jnp.where(idx < 11, n_b0, n_b1)).astype(jnp.int32)
        sti[...] = packed
        pltpu.sync_copy(sti, shi.at[sid])
        plsc.subcore_barrier()
        pltpu.sync_copy(shi, rdi)

        def cnt_comb(i, t):
            return t + rdi[i]
        accv = lax.fori_loop(0, NSUB, cnt_comb, zi)
        n_a_t = jnp.max(jnp.where(idx < 6, accv, 0))
        n_b0_t = jnp.max(jnp.where((idx >= 6) & (idx < 11), accv, 0))
        n_b1_t = jnp.max(jnp.where(idx >= 11, accv, 0))
        plsc.subcore_barrier()

        ok_a = n_a_t >= K
        lo1 = jnp.where(ok_a, cand_a, lo)
        n_b_t = jnp.where(ok_a, n_b1_t, n_b0_t)
        cand_b = lo1 | (one << bit_b)
        return jnp.where(n_b_t >= K, cand_b, lo1)

    lo = lax.fori_loop(0, NPAIR, round_body, lo0)

    cand1 = lo | jnp.uint32(1)
    cand2 = cand1 + jnp.uint32(1)
    v1 = jnp.full((LANES,), cand1)
    v2 = jnp.full((LANES,), cand2)

    def final_body(i, carry):
        ca1, ca2, s1, s2 = carry
        for j in range(4):
            sl = pl.ds((4 * i + j) * LANES, LANES)
            ku = lax.bitcast_convert_type(data[sl], jnp.uint32)
            m1_ = ku >= v1
            m2_ = ku >= v2
            ca1 = ca1 + jnp.where(m1_, 1, 0).astype(jnp.int32)
            ca2 = ca2 + jnp.where(m2_, 1, 0).astype(jnp.int32)
            ub = ku ^ jnp.where(ku >= _TOP, _TOP, _ALL)
            x = lax.bitcast_convert_type(ub, jnp.float32)
            e = jnp.exp((x - Mvec) * it_vec)
            s1 = s1 + jnp.where(m1_, e, jnp.float32(0))
            s2 = s2 + jnp.where(m2_, e, jnp.float32(0))
        return ca1, ca2, s1, s2

    ca1, ca2, s1, s2 = lax.fori_loop(
        0, NVREG // 4, final_body, (zi, zi, zf, zf))

    sti[...] = jnp.where(idx < 8, jnp.sum(ca1), jnp.sum(ca2)).astype(jnp.int32)
    stf[...] = s1
    pltpu.sync_copy(sti, shi.at[sid])
    pltpu.sync_copy(stf, shf.at[sid])
    stf[...] = s2
    pltpu.sync_copy(stf, shf.at[NSUB + sid])
    plsc.subcore_barrier()
    pltpu.sync_copy(shf, rdf)
    pltpu.sync_copy(shi, rdi)

    def fin_comb(i, carry):
        sv1, sv2, cv = carry
        return sv1 + rdf[i], sv2 + rdf[NSUB + i], cv + rdi[i]
    sv1, sv2, cv = lax.fori_loop(0, NSUB, fin_comb, (zf, zf, zi))
    cnt1_t = jnp.max(jnp.where(idx < 8, cv, 0))
    cnt2_t = jnp.max(jnp.where(idx >= 8, cv, 0))
    S1_t = jnp.sum(sv1)
    S2_t = jnp.sum(sv2)

    ok1 = cnt1_t >= K
    v = jnp.where(ok1, cand1, lo)
    cnt_gt = jnp.where(ok1, cnt2_t, cnt1_t)
    S_gt = jnp.where(ok1, S2_t, S1_t)

    mult = (K - cnt_gt).astype(jnp.float32)
    ub_v = v ^ jnp.where(v >= _TOP, _TOP, _ALL)
    v_f = lax.bitcast_convert_type(ub_v, jnp.float32)
    term_vec = jnp.exp((jnp.full((LANES,), v_f) - Mvec) * it_vec)
    S_full_vec = jnp.full((LANES,), S_gt) + term_vec * mult

    outv = jnp.where(idx == 0, S_full_vec, Mvec)

    @pl.when(sid == 0)
    def _():
        stf[...] = outv
        pltpu.sync_copy(stf, out_hbm)


def kernel(temperature, advantages):
    tp = temperature + 0.001
    inv_t = jnp.broadcast_to(1.0 / tp, (LANES,)).astype(jnp.float32)
    out = _sc_loss(inv_t, advantages)
    S = out[0]
    M = out[1]
    lse = M / tp + jnp.log(S)
    n = jnp.float32(K)
    loss = temperature * COEF_TEMP + temperature * (lse - jnp.log(n))
    return jnp.squeeze(loss)


# speedup vs baseline: 1.6153x; 1.6153x over previous
import functools

import jax
import jax.numpy as jnp
import numpy as np
from jax import lax
from jax.experimental import pallas as pl
from jax.experimental.pallas import tpu as pltpu
from jax.experimental.pallas import tpu_sc as plsc

COEF_TEMP = 0.0001
N = 1048576
K = N // 2
NSUB = 16
CHUNK = N // NSUB
LANES = 16
NVREG = CHUNK // LANES
NROUND = 30

_TOP = np.uint32(0x80000000)
_ALL = np.uint32(0xFFFFFFFF)


def _mesh():
    return plsc.VectorSubcoreMesh(
        core_axis_name="c", subcore_axis_name="s", num_cores=1)


@functools.partial(
    pl.kernel,
    out_type=jax.ShapeDtypeStruct((LANES,), jnp.float32),
    mesh=_mesh(),
    compiler_params=pltpu.CompilerParams(needs_layout_passes=False),
    scratch_types=[
        pltpu.VMEM((CHUNK,), jnp.float32),
        pltpu.VMEM((LANES,), jnp.float32),
        pltpu.VMEM((LANES,), jnp.int32),
        pltpu.VMEM((2 * NSUB, LANES), jnp.float32),
        pltpu.VMEM((NSUB, LANES), jnp.int32),
        pltpu.VMEM_SHARED((2 * NSUB, LANES), jnp.float32),
        pltpu.VMEM_SHARED((NSUB, LANES), jnp.int32),
    ],
)
def _sc_loss(inv_t_hbm, adv_hbm, out_hbm,
             data, stf, sti, rdf, rdi, shf, shi):
    sid = lax.axis_index("s")
    base = sid * CHUNK

    pltpu.sync_copy(adv_hbm.at[pl.ds(base, CHUNK)], data)
    pltpu.sync_copy(inv_t_hbm, stf)
    it_vec = stf[...]

    zi = jnp.zeros((LANES,), jnp.int32)
    zf = jnp.zeros((LANES,), jnp.float32)
    idx = lax.iota(jnp.int32, 16)

    def max_conv_body(i, acc):
        m0, m1, c0, c1 = acc
        for j in range(4):
            sl = pl.ds((4 * i + j) * LANES, LANES)
            x = data[sl]
            if j % 2 == 0:
                m0 = jnp.maximum(m0, x)
            else:
                m1 = jnp.maximum(m1, x)
            b = lax.bitcast_convert_type(x, jnp.int32)
            ku = lax.bitcast_convert_type(b, jnp.uint32) ^ jnp.where(
                b < 0, _ALL, _TOP)
            data[sl] = lax.bitcast_convert_type(ku, jnp.float32)
            if j % 2 == 0:
                c0 = c0 + plsc.all_reduce_population_count(ku >= _TOP)
            else:
                c1 = c1 + plsc.all_reduce_population_count(ku >= _TOP)
        return m0, m1, c0, c1

    ninf = jnp.full((LANES,), -jnp.inf, jnp.float32)
    m0, m1, c0, c1 = lax.fori_loop(
        0, NVREG // 4, max_conv_body, (ninf, ninf, zi, zi))
    stf[...] = jnp.maximum(m0, m1)
    sti[...] = c0 + c1
    pltpu.sync_copy(stf, shf.at[sid])
    pltpu.sync_copy(sti, shi.at[sid])
    plsc.subcore_barrier()
    pltpu.sync_copy(shf.at[pl.ds(0, NSUB)], rdf.at[pl.ds(0, NSUB)])
    pltpu.sync_copy(shi, rdi)

    def comb(i, carry):
        m, c = carry
        return jnp.maximum(m, rdf[i]), c + rdi[i]
    Mvec_all, cpos = lax.fori_loop(0, NSUB, comb, (ninf, zi))
    M = jnp.max(Mvec_all)
    Mvec = jnp.full((LANES,), M)
    tot_pos = jnp.max(cpos)
    lo0 = jnp.where(tot_pos >= K, jnp.uint32(_TOP), jnp.uint32(0))
    plsc.subcore_barrier()

    def round_body(r, lo):
        bit = jnp.uint32(30) - r.astype(jnp.uint32)
        cand = lo | (jnp.uint32(1) << bit)
        cand_v = jnp.full((LANES,), cand)

        def count_body(i, acc):
            a0, a1 = acc
            for j in range(8):
                sl = pl.ds((8 * i + j) * LANES, LANES)
                ku = lax.bitcast_convert_type(data[sl], jnp.uint32)
                pc = plsc.all_reduce_population_count(ku >= cand_v)
                if j % 2 == 0:
                    a0 = a0 + pc
                else:
                    a1 = a1 + pc
            return a0, a1

        a0, a1 = lax.fori_loop(0, NVREG // 8, count_body, (zi, zi))
        sti[...] = a0 + a1
        pltpu.sync_copy(sti, shi.at[sid])
        plsc.subcore_barrier()
        pltpu.sync_copy(shi, rdi)

        def cnt_comb(i, t):
            return t + rdi[i]
        total = jnp.max(lax.fori_loop(0, NSUB, cnt_comb, zi))
        plsc.subcore_barrier()
        return jnp.where(total >= K, cand, lo)

    lo = lax.fori_loop(0, NROUND, round_body, lo0)

    cand1 = lo | jnp.uint32(1)
    cand2 = cand1 + jnp.uint32(1)
    v1 = jnp.full((LANES,), cand1)
    v2 = jnp.full((LANES,), cand2)

    def final_body(i, carry):
        ca1, ca2, s1, s2 = carry
        for j in range(4):
            sl = pl.ds((4 * i + j) * LANES, LANES)
            ku = lax.bitcast_convert_type(data[sl], jnp.uint32)
            m1_ = ku >= v1
            m2_ = ku >= v2
            ca1 = ca1 + plsc.all_reduce_population_count(m1_)
            ca2 = ca2 + plsc.all_reduce_population_count(m2_)
            ub = ku ^ jnp.where(ku >= _TOP, _TOP, _ALL)
            x = lax.bitcast_convert_type(ub, jnp.float32)
            e = jnp.exp((x - Mvec) * it_vec)
            s1 = s1 + jnp.where(m1_, e, jnp.float32(0))
            s2 = s2 + jnp.where(m2_, e, jnp.float32(0))
        return ca1, ca2, s1, s2

    ca1, ca2, s1, s2 = lax.fori_loop(
        0, NVREG // 4, final_body, (zi, zi, zf, zf))

    sti[...] = jnp.where(idx < 8, jnp.max(ca1), jnp.max(ca2)).astype(jnp.int32)
    stf[...] = s1
    pltpu.sync_copy(sti, shi.at[sid])
    pltpu.sync_copy(stf, shf.at[sid])
    stf[...] = s2
    pltpu.sync_copy(stf, shf.at[NSUB + sid])
    plsc.subcore_barrier()
    pltpu.sync_copy(shf, rdf)
    pltpu.sync_copy(shi, rdi)

    def fin_comb(i, carry):
        sv1, sv2, cv = carry
        return sv1 + rdf[i], sv2 + rdf[NSUB + i], cv + rdi[i]
    sv1, sv2, cv = lax.fori_loop(0, NSUB, fin_comb, (zf, zf, zi))
    cnt1_t = jnp.max(jnp.where(idx < 8, cv, 0))
    cnt2_t = jnp.max(jnp.where(idx >= 8, cv, 0))
    S1_t = jnp.sum(sv1)
    S2_t = jnp.sum(sv2)

    ok1 = cnt1_t >= K
    v = jnp.where(ok1, cand1, lo)
    cnt_gt = jnp.where(ok1, cnt2_t, cnt1_t)
    S_gt = jnp.where(ok1, S2_t, S1_t)

    mult = (K - cnt_gt).astype(jnp.float32)
    ub_v = v ^ jnp.where(v >= _TOP, _TOP, _ALL)
    v_f = lax.bitcast_convert_type(ub_v, jnp.float32)
    term_vec = jnp.exp((jnp.full((LANES,), v_f) - Mvec) * it_vec)
    S_full_vec = jnp.full((LANES,), S_gt) + term_vec * mult

    outv = jnp.where(idx == 0, S_full_vec, Mvec)

    @pl.when(sid == 0)
    def _():
        stf[...] = outv
        pltpu.sync_copy(stf, out_hbm)


def kernel(temperature, advantages):
    tp = temperature + 0.001
    inv_t = jnp.broadcast_to(1.0 / tp, (LANES,)).astype(jnp.float32)
    out = _sc_loss(inv_t, advantages)
    S = out[0]
    M = out[1]
    lse = M / tp + jnp.log(S)
    n = jnp.float32(K)
    loss = temperature * COEF_TEMP + temperature * (lse - jnp.log(n))
    return jnp.squeeze(loss)


# speedup vs baseline: 1.6711x; 1.0345x over previous
import functools

import jax
import jax.numpy as jnp
import numpy as np
from jax import lax
from jax.experimental import pallas as pl
from jax.experimental.pallas import tpu as pltpu
from jax.experimental.pallas import tpu_sc as plsc

COEF_TEMP = 0.0001
N = 1048576
K = N // 2
NSUB = 16
CHUNK = N // NSUB
LANES = 16
NVREG = CHUNK // LANES
NROUND = 30

_TOP = np.uint32(0x80000000)
_ALL = np.uint32(0xFFFFFFFF)


def _mesh():
    return plsc.VectorSubcoreMesh(
        core_axis_name="c", subcore_axis_name="s", num_cores=1)


@functools.partial(
    pl.kernel,
    out_type=jax.ShapeDtypeStruct((LANES,), jnp.float32),
    mesh=_mesh(),
    compiler_params=pltpu.CompilerParams(needs_layout_passes=False),
    scratch_types=[
        pltpu.VMEM((CHUNK,), jnp.float32),
        pltpu.VMEM((LANES,), jnp.float32),
        pltpu.VMEM((LANES,), jnp.int32),
        pltpu.VMEM((NROUND * LANES,), jnp.int32),
        pltpu.VMEM((NSUB, LANES), jnp.float32),
        pltpu.VMEM((NSUB, LANES), jnp.int32),
        pltpu.VMEM_SHARED((NSUB, LANES), jnp.float32),
        pltpu.VMEM_SHARED((NSUB, LANES), jnp.int32),
        pltpu.VMEM_SHARED((NROUND * LANES,), jnp.int32),
    ],
)
def _sc_loss(inv_t_hbm, adv_hbm, out_hbm,
             data, stf, sti, zvm, rdf, rdi, shf, shi, shr):
    sid = lax.axis_index("s")
    base = sid * CHUNK

    pltpu.sync_copy(adv_hbm.at[pl.ds(base, CHUNK)], data)
    pltpu.sync_copy(inv_t_hbm, stf)
    it_vec = stf[...]

    zi = jnp.zeros((LANES,), jnp.int32)
    zf = jnp.zeros((LANES,), jnp.float32)

    @pl.when(sid == 0)
    def _():
        def zb(i, c):
            zvm[pl.ds(i * LANES, LANES)] = zi
            return c
        lax.fori_loop(0, NROUND, zb, 0)
        pltpu.sync_copy(zvm, shr)

    def max_conv_body(i, acc):
        m0, m1, c0, c1 = acc
        for j in range(4):
            sl = pl.ds((4 * i + j) * LANES, LANES)
            x = data[sl]
            if j % 2 == 0:
                m0 = jnp.maximum(m0, x)
            else:
                m1 = jnp.maximum(m1, x)
            b = lax.bitcast_convert_type(x, jnp.int32)
            ku = lax.bitcast_convert_type(b, jnp.uint32) ^ jnp.where(
                b < 0, _ALL, _TOP)
            data[sl] = lax.bitcast_convert_type(ku, jnp.float32)
            if j % 2 == 0:
                c0 = c0 + jnp.where(ku >= _TOP, 1, 0).astype(jnp.int32)
            else:
                c1 = c1 + jnp.where(ku >= _TOP, 1, 0).astype(jnp.int32)
        return m0, m1, c0, c1

    ninf = jnp.full((LANES,), -jnp.inf, jnp.float32)
    m0, m1, c0, c1 = lax.fori_loop(
        0, NVREG // 4, max_conv_body, (ninf, ninf, zi, zi))
    stf[...] = jnp.maximum(m0, m1)
    sti[...] = jnp.full((LANES,), jnp.sum(c0 + c1), jnp.int32)
    pltpu.sync_copy(stf, shf.at[sid])
    pltpu.sync_copy(sti, shi.at[sid])
    plsc.subcore_barrier()
    pltpu.sync_copy(shf, rdf)
    pltpu.sync_copy(shi, rdi)

    def comb(i, carry):
        m, c = carry
        return jnp.maximum(m, rdf[i]), c + rdi[i]
    Mvec_all, cpos = lax.fori_loop(0, NSUB, comb, (ninf, zi))
    M = jnp.max(Mvec_all)
    Mvec = jnp.full((LANES,), M)
    tot_pos = jnp.max(cpos)
    lo0 = jnp.where(tot_pos >= K, jnp.uint32(_TOP), jnp.uint32(0))

    def round_body(r, lo):
        bit = jnp.uint32(30) - r.astype(jnp.uint32)
        cand = lo | (jnp.uint32(1) << bit)
        cand_v = jnp.full((LANES,), cand)

        def count_body(i, acc):
            a0, a1 = acc
            for j in range(16):
                sl = pl.ds((16 * i + j) * LANES, LANES)
                ku = lax.bitcast_convert_type(data[sl], jnp.uint32)
                m = jnp.where(ku >= cand_v, 1, 0).astype(jnp.int32)
                if j % 2 == 0:
                    a0 = a0 + m
                else:
                    a1 = a1 + m
            return a0, a1

        a0, a1 = lax.fori_loop(0, NVREG // 16, count_body, (zi, zi))
        cnt = jnp.sum(a0 + a1)
        sti[...] = jnp.full((LANES,), cnt, jnp.int32)
        row_idx = r * LANES + lax.iota(jnp.int32, LANES)
        pltpu.sync_copy(sti, shr.at[row_idx], add=True)
        plsc.subcore_barrier()
        pltpu.sync_copy(shr.at[pl.ds(r * LANES, LANES)], sti)
        total = jnp.max(sti[...])
        return jnp.where(total >= K, cand, lo)

    lo = lax.fori_loop(0, NROUND, round_body, lo0)

    idx = lax.iota(jnp.int32, 16)
    cand1 = lo | jnp.uint32(1)
    cand2 = cand1 + jnp.uint32(1)
    v1 = jnp.full((LANES,), cand1)
    v2 = jnp.full((LANES,), cand2)

    def final_body(i, carry):
        ca1, ca2, s1, s2 = carry
        for j in range(4):
            sl = pl.ds((4 * i + j) * LANES, LANES)
            ku = lax.bitcast_convert_type(data[sl], jnp.uint32)
            m1_ = ku >= v1
            m2_ = ku >= v2
            ca1 = ca1 + jnp.where(m1_, 1, 0).astype(jnp.int32)
            ca2 = ca2 + jnp.where(m2_, 1, 0).astype(jnp.int32)
            ub = ku ^ jnp.where(ku >= _TOP, _TOP, _ALL)
            x = lax.bitcast_convert_type(ub, jnp.float32)
            e = jnp.exp((x - Mvec) * it_vec)
            s1 = s1 + jnp.where(m1_, e, jnp.float32(0))
            s2 = s2 + jnp.where(m2_, e, jnp.float32(0))
        return ca1, ca2, s1, s2

    ca1, ca2, s1, s2 = lax.fori_loop(
        0, NVREG // 4, final_body, (zi, zi, zf, zf))

    sti[...] = jnp.where(idx < 8, jnp.sum(ca1), jnp.sum(ca2)).astype(jnp.int32)
    stf[...] = jnp.where(idx < 8, jnp.sum(s1), jnp.sum(s2))
    pltpu.sync_copy(sti, shi.at[sid])
    pltpu.sync_copy(stf, shf.at[sid])
    plsc.subcore_barrier()
    pltpu.sync_copy(shf, rdf)
    pltpu.sync_copy(shi, rdi)

    def fin_comb(i, carry):
        sv, cv = carry
        return sv + rdf[i], cv + rdi[i]
    sv, cv = lax.fori_loop(0, NSUB, fin_comb, (zf, zi))
    cnt1_t = jnp.max(jnp.where(idx < 8, cv, 0))
    cnt2_t = jnp.max(jnp.where(idx >= 8, cv, 0))
    S1_t = jnp.max(jnp.where(idx < 8, sv, jnp.float32(0)))
    S2_t = jnp.max(jnp.where(idx >= 8, sv, jnp.float32(0)))

    ok1 = cnt1_t >= K
    v = jnp.where(ok1, cand1, lo)
    cnt_gt = jnp.where(ok1, cnt2_t, cnt1_t)
    S_gt = jnp.where(ok1, S2_t, S1_t)

    mult = (K - cnt_gt).astype(jnp.float32)
    ub_v = v ^ jnp.where(v >= _TOP, _TOP, _ALL)
    v_f = lax.bitcast_convert_type(ub_v, jnp.float32)
    term_vec = jnp.exp((jnp.full((LANES,), v_f) - Mvec) * it_vec)
    S_full_vec = jnp.full((LANES,), S_gt) + term_vec * mult

    outv = jnp.where(idx == 0, S_full_vec, Mvec)

    @pl.when(sid == 0)
    def _():
        stf[...] = outv
        pltpu.sync_copy(stf, out_hbm)


def kernel(temperature, advantages):
    tp = temperature + 0.001
    inv_t = jnp.broadcast_to(1.0 / tp, (LANES,)).astype(jnp.float32)
    out = _sc_loss(inv_t, advantages)
    S = out[0]
    M = out[1]
    lse = M / tp + jnp.log(S)
    n = jnp.float32(K)
    loss = temperature * COEF_TEMP + temperature * (lse - jnp.log(n))
    return jnp.squeeze(loss)


# speedup vs baseline: 1.7016x; 1.0183x over previous
import functools

import jax
import jax.numpy as jnp
import numpy as np
from jax import lax
from jax.experimental import pallas as pl
from jax.experimental.pallas import tpu as pltpu
from jax.experimental.pallas import tpu_sc as plsc

COEF_TEMP = 0.0001
N = 1048576
K = N // 2
NSUB = 16
CHUNK = N // NSUB
LANES = 16
NVREG = CHUNK // LANES
NROUND = 31

_TOP = np.uint32(0x80000000)
_ALL = np.uint32(0xFFFFFFFF)


def _mesh():
    return plsc.VectorSubcoreMesh(
        core_axis_name="c", subcore_axis_name="s", num_cores=1)


@functools.partial(
    pl.kernel,
    out_type=jax.ShapeDtypeStruct((LANES,), jnp.float32),
    mesh=_mesh(),
    compiler_params=pltpu.CompilerParams(needs_layout_passes=False),
    scratch_types=[
        pltpu.VMEM((CHUNK,), jnp.float32),
        pltpu.VMEM((LANES,), jnp.float32),
        pltpu.VMEM((LANES,), jnp.int32),
        pltpu.VMEM((NROUND * LANES,), jnp.int32),
        pltpu.VMEM((NSUB, LANES), jnp.float32),
        pltpu.VMEM((NSUB, LANES), jnp.int32),
        pltpu.VMEM_SHARED((NSUB, LANES), jnp.float32),
        pltpu.VMEM_SHARED((NSUB, LANES), jnp.int32),
        pltpu.VMEM_SHARED((NROUND * LANES,), jnp.int32),
    ],
)
def _sc_loss(inv_t_hbm, adv_hbm, out_hbm,
             data, stf, sti, zvm, rdf, rdi, shf, shi, shr):
    sid = lax.axis_index("s")
    base = sid * CHUNK

    pltpu.sync_copy(adv_hbm.at[pl.ds(base, CHUNK)], data)
    pltpu.sync_copy(inv_t_hbm, stf)
    it_vec = stf[...]

    zi = jnp.zeros((LANES,), jnp.int32)
    zf = jnp.zeros((LANES,), jnp.float32)

    @pl.when(sid == 0)
    def _():
        def zb(i, c):
            zvm[pl.ds(i * LANES, LANES)] = zi
            return c
        lax.fori_loop(0, NROUND, zb, 0)
        pltpu.sync_copy(zvm, shr)

    def max_conv_body(i, acc):
        m0, m1, c0, c1 = acc
        for j in range(4):
            sl = pl.ds((4 * i + j) * LANES, LANES)
            x = data[sl]
            if j % 2 == 0:
                m0 = jnp.maximum(m0, x)
            else:
                m1 = jnp.maximum(m1, x)
            b = lax.bitcast_convert_type(x, jnp.int32)
            ku = lax.bitcast_convert_type(b, jnp.uint32) ^ jnp.where(
                b < 0, _ALL, _TOP)
            data[sl] = lax.bitcast_convert_type(ku, jnp.float32)
            if j % 2 == 0:
                c0 = c0 + jnp.where(ku >= _TOP, 1, 0).astype(jnp.int32)
            else:
                c1 = c1 + jnp.where(ku >= _TOP, 1, 0).astype(jnp.int32)
        return m0, m1, c0, c1

    ninf = jnp.full((LANES,), -jnp.inf, jnp.float32)
    m0, m1, c0, c1 = lax.fori_loop(
        0, NVREG // 4, max_conv_body, (ninf, ninf, zi, zi))
    stf[...] = jnp.maximum(m0, m1)
    sti[...] = jnp.full((LANES,), jnp.sum(c0 + c1), jnp.int32)
    pltpu.sync_copy(stf, shf.at[sid])
    pltpu.sync_copy(sti, shi.at[sid])
    plsc.subcore_barrier()
    pltpu.sync_copy(shf, rdf)
    pltpu.sync_copy(shi, rdi)

    def comb(i, carry):
        m, c = carry
        return jnp.maximum(m, rdf[i]), c + rdi[i]
    Mvec_all, cpos = lax.fori_loop(0, NSUB, comb, (ninf, zi))
    M = jnp.max(Mvec_all)
    Mvec = jnp.full((LANES,), M)
    tot_pos = jnp.max(cpos)
    lo0 = jnp.where(tot_pos >= K, jnp.uint32(_TOP), jnp.uint32(0))

    def round_body(r, lo):
        bit = jnp.uint32(30) - r.astype(jnp.uint32)
        cand = lo | (jnp.uint32(1) << bit)
        cand_v = jnp.full((LANES,), cand)

        def count_body(i, acc):
            a0, a1 = acc
            for j in range(8):
                sl = pl.ds((8 * i + j) * LANES, LANES)
                ku = lax.bitcast_convert_type(data[sl], jnp.uint32)
                m = jnp.where(ku >= cand_v, 1, 0).astype(jnp.int32)
                if j % 2 == 0:
                    a0 = a0 + m
                else:
                    a1 = a1 + m
            return a0, a1

        a0, a1 = lax.fori_loop(0, NVREG // 8, count_body, (zi, zi))
        cnt = jnp.sum(a0 + a1)
        sti[...] = jnp.full((LANES,), cnt, jnp.int32)
        row_idx = r * LANES + lax.iota(jnp.int32, LANES)
        pltpu.sync_copy(sti, shr.at[row_idx], add=True)
        plsc.subcore_barrier()
        pltpu.sync_copy(shr.at[pl.ds(r * LANES, LANES)], sti)
        total = jnp.max(sti[...])
        return jnp.where(total >= K, cand, lo)

    v = lax.fori_loop(0, NROUND, round_body, lo0)

    v_v = jnp.full((LANES,), v)

    def final_body(i, carry):
        cacc, sacc = carry
        for j in range(4):
            sl = pl.ds((4 * i + j) * LANES, LANES)
            ku = lax.bitcast_convert_type(data[sl], jnp.uint32)
            gt = ku > v_v
            cacc = cacc + jnp.where(gt, 1, 0).astype(jnp.int32)
            ub = ku ^ jnp.where(ku >= _TOP, _TOP, _ALL)
            x = lax.bitcast_convert_type(ub, jnp.float32)
            e = jnp.exp((x - Mvec) * it_vec)
            sacc = sacc + jnp.where(gt, e, jnp.float32(0))
        return cacc, sacc

    cacc, sacc = lax.fori_loop(0, NVREG // 4, final_body, (zi, zf))

    stf[...] = sacc
    sti[...] = jnp.full((LANES,), jnp.sum(cacc), jnp.int32)
    pltpu.sync_copy(stf, shf.at[sid])
    pltpu.sync_copy(sti, shi.at[sid])
    plsc.subcore_barrier()
    pltpu.sync_copy(shf, rdf)
    pltpu.sync_copy(shi, rdi)

    def fin_comb(i, carry):
        sv, cv = carry
        return sv + rdf[i], cv + rdi[i]
    sv, cv = lax.fori_loop(0, NSUB, fin_comb, (zf, zi))
    S_gt = jnp.sum(sv)
    cnt_gt = jnp.max(cv)

    mult = (K - cnt_gt).astype(jnp.float32)
    ub_v = v ^ jnp.where(v >= _TOP, _TOP, _ALL)
    v_f = lax.bitcast_convert_type(ub_v, jnp.float32)
    term_vec = jnp.exp((jnp.full((LANES,), v_f) - Mvec) * it_vec)
    S_full_vec = jnp.full((LANES,), S_gt) + term_vec * mult

    idx = lax.iota(jnp.int32, 16)
    outv = jnp.where(idx == 0, S_full_vec, Mvec)

    @pl.when(sid == 0)
    def _():
        stf[...] = outv
        pltpu.sync_copy(stf, out_hbm)


def kernel(temperature, advantages):
    tp = temperature + 0.001
    inv_t = jnp.broadcast_to(1.0 / tp, (LANES,)).astype(jnp.float32)
    out = _sc_loss(inv_t, advantages)
    S = out[0]
    M = out[1]
    lse = M / tp + jnp.log(S)
    n = jnp.float32(K)
    loss = temperature * COEF_TEMP + temperature * (lse - jnp.log(n))
    return jnp.squeeze(loss)
